# baseline (speedup 1.0000x reference)
import functools

import jax
import jax.numpy as jnp
from jax import lax
from jax.experimental import pallas as pl
from jax.experimental.pallas import tpu as pltpu
from jax.experimental.pallas import tpu_sc as plsc

N = 10000
NPAD = 10240
E = 320000
NBLK = 79
NW = 32
EPAD = NW * NBLK * 128
F = 128
C = 32
PAD_SRC = 10000
PAD_DST = 10200
ROWS_PER_SUB = NPAD // 16

_MESH = plsc.VectorSubcoreMesh(core_axis_name="c", subcore_axis_name="s")


@functools.partial(
    pl.kernel,
    out_type=jax.ShapeDtypeStruct((2, NPAD, F), jnp.float32),
    mesh=_MESH,
    scratch_types=[
        pltpu.VMEM((NBLK, 128), jnp.int32),
        pltpu.VMEM((NBLK, 128), jnp.int32),
        pltpu.VMEM((128, F), jnp.float32),
        pltpu.VMEM_SHARED((NPAD, F), jnp.float32),
        pltpu.SemaphoreType.DMA,
    ],
)
def _agg_kernel(s_hbm, srcb, dstb, zeros, out, sidx, didx, rows, acc, sem):
    c = lax.axis_index("c")
    s = lax.axis_index("s")
    wid = s * 2 + c
    base = s * ROWS_PER_SUB
    pltpu.sync_copy(zeros.at[pl.ds(base, ROWS_PER_SUB)],
                    acc.at[pl.ds(base, ROWS_PER_SUB)])
    pltpu.sync_copy(srcb.at[wid], sidx)
    pltpu.sync_copy(dstb.at[wid], didx)
    plsc.subcore_barrier()

    def body(j, carry):
        pltpu.async_copy(s_hbm.at[sidx.at[j]], rows, sem).wait()
        pltpu.sync_copy(rows, acc.at[didx.at[j]], add=True)
        return carry

    lax.fori_loop(0, NBLK, body, 0)
    plsc.subcore_barrier()
    pltpu.sync_copy(acc.at[pl.ds(base, ROWS_PER_SUB)],
                    out.at[c, pl.ds(base, ROWS_PER_SUB)])


def _prep_body(dego_ref, degi_ref, nsb_ref, ndb_ref):
    do = dego_ref[0, :, 0:1] + dego_ref[1, :, 0:1]
    di = degi_ref[0, :, 0:1] + degi_ref[1, :, 0:1]
    ns = lax.rsqrt(jnp.maximum(do, 1.0))
    nd = lax.rsqrt(jnp.maximum(di, 1.0))
    nsb_ref[...] = jnp.broadcast_to(ns, (NPAD, F))
    ndb_ref[...] = jnp.broadcast_to(nd, (NPAD, F))


_prep_call = pl.pallas_call(
    _prep_body,
    out_shape=(jax.ShapeDtypeStruct((NPAD, F), jnp.float32),
               jax.ShapeDtypeStruct((NPAD, F), jnp.float32)),
)


def _l1_body(x_ref, w_ref, nsb_ref, out_ref):
    out_ref[...] = jnp.dot(x_ref[...], w_ref[...],
                           preferred_element_type=jnp.float32) * nsb_ref[...]


_l1_call = pl.pallas_call(
    _l1_body,
    grid=(NPAD // 128,),
    in_specs=[
        pl.BlockSpec((128, F), lambda i: (i, 0)),
        pl.BlockSpec((F, F), lambda i: (0, 0)),
        pl.BlockSpec((128, F), lambda i: (i, 0)),
    ],
    out_specs=pl.BlockSpec((128, F), lambda i: (i, 0)),
    out_shape=jax.ShapeDtypeStruct((NPAD, F), jnp.float32),
)


def _layer_body(p_ref, ndb_ref, b_ref, w_ref, nsb_ref, out_ref):
    h = jnp.maximum((p_ref[0] + p_ref[1]) * ndb_ref[...] + b_ref[...], 0.0)
    out_ref[...] = jnp.dot(h, w_ref[...],
                           preferred_element_type=jnp.float32) * nsb_ref[...]


_layer_call = pl.pallas_call(
    _layer_body,
    grid=(NPAD // 128,),
    in_specs=[
        pl.BlockSpec((2, 128, F), lambda i: (0, i, 0)),
        pl.BlockSpec((128, F), lambda i: (i, 0)),
        pl.BlockSpec((1, F), lambda i: (0, 0)),
        pl.BlockSpec((F, F), lambda i: (0, 0)),
        pl.BlockSpec((128, F), lambda i: (i, 0)),
    ],
    out_specs=pl.BlockSpec((128, F), lambda i: (i, 0)),
    out_shape=jax.ShapeDtypeStruct((NPAD, F), jnp.float32),
)


def _final_body(p_ref, ndb_ref, b_ref, wl_ref, bl_ref, out_ref):
    h = jnp.maximum((p_ref[0] + p_ref[1]) * ndb_ref[...] + b_ref[...], 0.0)
    rows = lax.broadcasted_iota(jnp.int32, (NPAD, F), 0)
    h = jnp.where(rows < N, h, 0.0)
    pooled = jnp.sum(h, axis=0, keepdims=True) * (1.0 / N)
    logits = jnp.dot(pooled, wl_ref[...],
                     preferred_element_type=jnp.float32) + bl_ref[...]
    m = jnp.max(logits, axis=1, keepdims=True)
    ex = jnp.exp(logits - m)
    out_ref[...] = (logits - m) - jnp.log(jnp.sum(ex, axis=1, keepdims=True))


_final_call = pl.pallas_call(
    _final_body,
    out_shape=jax.ShapeDtypeStruct((1, C), jnp.float32),
)


def kernel(features, edge_index, W1, b1, W2, b2, W3, b3, Wl, bl):
    ei = edge_index.astype(jnp.int32)
    src = jnp.concatenate(
        [ei[0], jnp.full((EPAD - E,), PAD_SRC, jnp.int32)]).reshape(NW, NBLK, 128)
    dst = jnp.concatenate(
        [ei[1], jnp.full((EPAD - E,), PAD_DST, jnp.int32)]).reshape(NW, NBLK, 128)
    x_p = jnp.pad(features, ((0, NPAD - N), (0, 0)))
    zeros = jnp.zeros((NPAD, F), jnp.float32)
    s0 = jnp.zeros((NPAD, F), jnp.float32).at[:, 0].set(1.0)
    zidx = jnp.zeros((NW, NBLK, 128), jnp.int32)
    dego = _agg_kernel(s0, zidx, src, zeros)
    degi = _agg_kernel(s0, zidx, dst, zeros)
    nsb, ndb = _prep_call(dego, degi)

    s1 = _l1_call(x_p, W1, nsb)
    p1 = _agg_kernel(s1, src, dst, zeros)
    s2 = _layer_call(p1, ndb, b1.reshape(1, F), W2, nsb)
    p2 = _agg_kernel(s2, src, dst, zeros)
    s3 = _layer_call(p2, ndb, b2.reshape(1, F), W3, nsb)
    p3 = _agg_kernel(s3, src, dst, zeros)
    return _final_call(p3, ndb, b3.reshape(1, F), Wl, bl.reshape(1, C))


# speedup vs baseline: 13.5909x; 13.5909x over previous
import functools

import jax
import jax.numpy as jnp
from jax import lax
from jax.experimental import pallas as pl
from jax.experimental.pallas import tpu as pltpu
from jax.experimental.pallas import tpu_sc as plsc

N = 10000
NPAD = 10240
E = 320000
NBLK = 79
NW = 32
EPAD = NW * NBLK * 128
F = 128
C = 32
PAD_SRC = 10000
PAD_DST = 10200
ROWS_PER_SUB = NPAD // 16

_MESH = plsc.VectorSubcoreMesh(core_axis_name="c", subcore_axis_name="s")


@functools.partial(
    pl.kernel,
    out_type=jax.ShapeDtypeStruct((2, NPAD, F), jnp.float32),
    mesh=_MESH,
    scratch_types=[
        pltpu.VMEM((NBLK, 128), jnp.int32),
        pltpu.VMEM((NBLK, 128), jnp.int32),
        pltpu.VMEM((128, F), jnp.float32),
        pltpu.VMEM_SHARED((NPAD, F), jnp.float32),
        pltpu.SemaphoreType.DMA,
    ],
)
def _agg_kernel(s_hbm, srcb, dstb, zeros, out, sidx, didx, rows, acc, sem):
    c = lax.axis_index("c")
    s = lax.axis_index("s")
    wid = s * 2 + c
    base = s * ROWS_PER_SUB
    pltpu.sync_copy(zeros.at[pl.ds(base, ROWS_PER_SUB)],
                    acc.at[pl.ds(base, ROWS_PER_SUB)])
    pltpu.sync_copy(srcb.at[wid], sidx)
    pltpu.sync_copy(dstb.at[wid], didx)
    plsc.subcore_barrier()

    def body(j, carry):
        pltpu.async_copy(s_hbm.at[sidx.at[j]], rows, sem).wait()
        pltpu.sync_copy(rows, acc.at[didx.at[j]], add=True)
        return carry

    lax.fori_loop(0, NBLK, body, 0)
    plsc.subcore_barrier()
    pltpu.sync_copy(acc.at[pl.ds(base, ROWS_PER_SUB)],
                    out.at[c, pl.ds(base, ROWS_PER_SUB)])


def _prep_body(dego_ref, degi_ref, nsb_ref, ndb_ref):
    do = dego_ref[0, :, 0:1] + dego_ref[1, :, 0:1]
    di = degi_ref[0, :, 0:1] + degi_ref[1, :, 0:1]
    ns = lax.rsqrt(jnp.maximum(do, 1.0))
    nd = lax.rsqrt(jnp.maximum(di, 1.0))
    nsb_ref[...] = jnp.broadcast_to(ns, (NPAD, F))
    ndb_ref[...] = jnp.broadcast_to(nd, (NPAD, F))


_prep_call = pl.pallas_call(
    _prep_body,
    out_shape=(jax.ShapeDtypeStruct((NPAD, F), jnp.float32),
               jax.ShapeDtypeStruct((NPAD, F), jnp.float32)),
)


def _l1_body(x_ref, w_ref, nsb_ref, out_ref):
    out_ref[...] = jnp.dot(x_ref[...], w_ref[...],
                           preferred_element_type=jnp.float32) * nsb_ref[...]


_l1_call = pl.pallas_call(
    _l1_body,
    grid=(NPAD // 128,),
    in_specs=[
        pl.BlockSpec((128, F), lambda i: (i, 0)),
        pl.BlockSpec((F, F), lambda i: (0, 0)),
        pl.BlockSpec((128, F), lambda i: (i, 0)),
    ],
    out_specs=pl.BlockSpec((128, F), lambda i: (i, 0)),
    out_shape=jax.ShapeDtypeStruct((NPAD, F), jnp.float32),
)


def _layer_body(p_ref, ndb_ref, b_ref, w_ref, nsb_ref, out_ref):
    h = jnp.maximum((p_ref[0] + p_ref[1]) * ndb_ref[...] + b_ref[...], 0.0)
    out_ref[...] = jnp.dot(h, w_ref[...],
                           preferred_element_type=jnp.float32) * nsb_ref[...]


_layer_call = pl.pallas_call(
    _layer_body,
    grid=(NPAD // 128,),
    in_specs=[
        pl.BlockSpec((2, 128, F), lambda i: (0, i, 0)),
        pl.BlockSpec((128, F), lambda i: (i, 0)),
        pl.BlockSpec((1, F), lambda i: (0, 0)),
        pl.BlockSpec((F, F), lambda i: (0, 0)),
        pl.BlockSpec((128, F), lambda i: (i, 0)),
    ],
    out_specs=pl.BlockSpec((128, F), lambda i: (i, 0)),
    out_shape=jax.ShapeDtypeStruct((NPAD, F), jnp.float32),
)


def _final_body(p_ref, ndb_ref, b_ref, wl_ref, bl_ref, out_ref):
    h = jnp.maximum((p_ref[0] + p_ref[1]) * ndb_ref[...] + b_ref[...], 0.0)
    rows = lax.broadcasted_iota(jnp.int32, (NPAD, F), 0)
    h = jnp.where(rows < N, h, 0.0)
    pooled = jnp.sum(h, axis=0, keepdims=True) * (1.0 / N)
    logits = jnp.dot(pooled, wl_ref[...],
                     preferred_element_type=jnp.float32) + bl_ref[...]
    m = jnp.max(logits, axis=1, keepdims=True)
    ex = jnp.exp(logits - m)
    out_ref[...] = (logits - m) - jnp.log(jnp.sum(ex, axis=1, keepdims=True))


_final_call = pl.pallas_call(
    _final_body,
    out_shape=jax.ShapeDtypeStruct((1, C), jnp.float32),
)


def kernel(features, edge_index, W1, b1, W2, b2, W3, b3, Wl, bl):
    ei = edge_index.astype(jnp.int32)
    src = jnp.concatenate(
        [ei[0], jnp.full((EPAD - E,), PAD_SRC, jnp.int32)]).reshape(NW, NBLK, 128)
    dst = jnp.concatenate(
        [ei[1], jnp.full((EPAD - E,), PAD_DST, jnp.int32)]).reshape(NW, NBLK, 128)
    x_p = jnp.pad(features, ((0, NPAD - N), (0, 0)))
    zeros = jnp.zeros((NPAD, F), jnp.float32)
    s0 = jnp.zeros((NPAD, F), jnp.float32).at[:, 0].set(1.0)
    dego = _agg_kernel(s0, src, src, zeros)
    degi = _agg_kernel(s0, dst, dst, zeros)
    nsb, ndb = _prep_call(dego, degi)

    s1 = _l1_call(x_p, W1, nsb)
    p1 = _agg_kernel(s1, src, dst, zeros)
    s2 = _layer_call(p1, ndb, b1.reshape(1, F), W2, nsb)
    p2 = _agg_kernel(s2, src, dst, zeros)
    s3 = _layer_call(p2, ndb, b2.reshape(1, F), W3, nsb)
    p3 = _agg_kernel(s3, src, dst, zeros)
    return _final_call(p3, ndb, b3.reshape(1, F), Wl, bl.reshape(1, C))


# speedup vs baseline: 35.4525x; 2.6086x over previous
import functools

import jax
import jax.numpy as jnp
from jax import lax
from jax.experimental import pallas as pl
from jax.experimental.pallas import tpu as pltpu
from jax.experimental.pallas import tpu_sc as plsc

N = 10000
NPAD = 10112
E = 320000
NBLK = 80
BW = 128
NW = 32
EPAD = NW * NBLK * BW
F = 128
C = 32
ROWS_PER_SUB = NPAD // 16

_MESH = plsc.VectorSubcoreMesh(core_axis_name="c", subcore_axis_name="s")


@functools.partial(
    pl.kernel,
    out_type=jax.ShapeDtypeStruct((2, NPAD, F), jnp.float32),
    mesh=_MESH,
    scratch_types=[
        pltpu.VMEM((2, BW), jnp.int32),
        pltpu.VMEM((2, BW), jnp.int32),
        pltpu.VMEM((BW, F), jnp.float32),
        pltpu.VMEM((BW, F), jnp.float32),
        pltpu.VMEM((16,), jnp.int32),
        pltpu.VMEM_SHARED((NPAD, F), jnp.float32),
        pltpu.SemaphoreType.DMA,
        pltpu.SemaphoreType.DMA,
        pltpu.SemaphoreType.DMA,
        pltpu.SemaphoreType.DMA,
    ],
)
def _agg_kernel(s_hbm, eidx, va_hbm, vb_hbm, zeros, flag, out,
                ibuf0, ibuf1, rows0, rows1, flagv, acc,
                semi0, semi1, sem0, sem1):
    c = lax.axis_index("c")
    s = lax.axis_index("s")
    wid = s * 2 + c
    base = s * ROWS_PER_SUB
    pltpu.sync_copy(zeros.at[pl.ds(base, ROWS_PER_SUB)],
                    acc.at[pl.ds(base, ROWS_PER_SUB)])
    pltpu.sync_copy(flag, flagv)
    mode = flagv[...][0]
    plsc.subcore_barrier()

    pltpu.async_copy(eidx.at[wid, 0], ibuf0, semi0)
    pltpu.async_copy(eidx.at[wid, 1], ibuf1, semi1)

    @pl.when(mode == 0)
    def _layer():
        pltpu.make_async_copy(eidx.at[wid, 0], ibuf0, semi0).wait()
        pltpu.async_copy(s_hbm.at[ibuf0.at[0]], rows0, sem0)

        def body(p, carry):
            j = 2 * p
            pltpu.make_async_copy(eidx.at[wid, j + 1], ibuf1, semi1).wait()
            pltpu.make_async_copy(s_hbm.at[ibuf0.at[0]], rows0, sem0).wait()
            g1 = pltpu.async_copy(s_hbm.at[ibuf1.at[0]], rows1, sem1)
            pltpu.sync_copy(rows0, acc.at[ibuf0.at[1]], add=True)

            @pl.when(j + 2 < NBLK)
            def _():
                pltpu.async_copy(eidx.at[wid, j + 2], ibuf0, semi0)

            g1.wait()

            @pl.when(j + 2 < NBLK)
            def _():
                pltpu.make_async_copy(eidx.at[wid, j + 2], ibuf0, semi0).wait()
                pltpu.async_copy(s_hbm.at[ibuf0.at[0]], rows0, sem0)

            pltpu.sync_copy(rows1, acc.at[ibuf1.at[1]], add=True)

            @pl.when(j + 3 < NBLK)
            def _():
                pltpu.async_copy(eidx.at[wid, j + 3], ibuf1, semi1)

            return carry

        lax.fori_loop(0, NBLK // 2, body, 0)

    @pl.when(mode == 1)
    def _deg():
        pltpu.sync_copy(va_hbm, rows0)
        pltpu.sync_copy(vb_hbm, rows1)

        def body(p, carry):
            j = 2 * p
            pltpu.make_async_copy(eidx.at[wid, j], ibuf0, semi0).wait()
            pltpu.sync_copy(rows0, acc.at[ibuf0.at[0]], add=True)
            pltpu.sync_copy(rows1, acc.at[ibuf0.at[1]], add=True)

            @pl.when(j + 2 < NBLK)
            def _():
                pltpu.async_copy(eidx.at[wid, j + 2], ibuf0, semi0)

            pltpu.make_async_copy(eidx.at[wid, j + 1], ibuf1, semi1).wait()
            pltpu.sync_copy(rows0, acc.at[ibuf1.at[0]], add=True)
            pltpu.sync_copy(rows1, acc.at[ibuf1.at[1]], add=True)

            @pl.when(j + 3 < NBLK)
            def _():
                pltpu.async_copy(eidx.at[wid, j + 3], ibuf1, semi1)

            return carry

        lax.fori_loop(0, NBLK // 2, body, 0)

    plsc.subcore_barrier()
    pltpu.sync_copy(acc.at[pl.ds(base, ROWS_PER_SUB)],
                    out.at[c, pl.ds(base, ROWS_PER_SUB)])


def _prep_body(deg_ref, nsb_ref, ndb_ref):
    d = deg_ref[0] + deg_ref[1]
    ns = lax.rsqrt(jnp.maximum(d[:, 0:1], 1.0))
    nd = lax.rsqrt(jnp.maximum(d[:, 1:2], 1.0))
    nsb_ref[...] = jnp.broadcast_to(ns, (NPAD, F))
    ndb_ref[...] = jnp.broadcast_to(nd, (NPAD, F))


_prep_call = pl.pallas_call(
    _prep_body,
    out_shape=(jax.ShapeDtypeStruct((NPAD, F), jnp.float32),
               jax.ShapeDtypeStruct((NPAD, F), jnp.float32)),
)


def _l1_body(x_ref, w_ref, nsb_ref, out_ref):
    out_ref[...] = jnp.dot(x_ref[...], w_ref[...],
                           preferred_element_type=jnp.float32) * nsb_ref[...]


_l1_call = pl.pallas_call(
    _l1_body,
    grid=(NPAD // 128,),
    in_specs=[
        pl.BlockSpec((128, F), lambda i: (i, 0)),
        pl.BlockSpec((F, F), lambda i: (0, 0)),
        pl.BlockSpec((128, F), lambda i: (i, 0)),
    ],
    out_specs=pl.BlockSpec((128, F), lambda i: (i, 0)),
    out_shape=jax.ShapeDtypeStruct((NPAD, F), jnp.float32),
)


def _layer_body(p_ref, ndb_ref, b_ref, w_ref, nsb_ref, out_ref):
    h = jnp.maximum((p_ref[0] + p_ref[1]) * ndb_ref[...] + b_ref[...], 0.0)
    out_ref[...] = jnp.dot(h, w_ref[...],
                           preferred_element_type=jnp.float32) * nsb_ref[...]


_layer_call = pl.pallas_call(
    _layer_body,
    grid=(NPAD // 128,),
    in_specs=[
        pl.BlockSpec((2, 128, F), lambda i: (0, i, 0)),
        pl.BlockSpec((128, F), lambda i: (i, 0)),
        pl.BlockSpec((1, F), lambda i: (0, 0)),
        pl.BlockSpec((F, F), lambda i: (0, 0)),
        pl.BlockSpec((128, F), lambda i: (i, 0)),
    ],
    out_specs=pl.BlockSpec((128, F), lambda i: (i, 0)),
    out_shape=jax.ShapeDtypeStruct((NPAD, F), jnp.float32),
)


def _final_body(p_ref, ndb_ref, b_ref, wl_ref, bl_ref, out_ref):
    h = jnp.maximum((p_ref[0] + p_ref[1]) * ndb_ref[...] + b_ref[...], 0.0)
    rows = lax.broadcasted_iota(jnp.int32, (NPAD, F), 0)
    h = jnp.where(rows < N, h, 0.0)
    pooled = jnp.sum(h, axis=0, keepdims=True) * (1.0 / N)
    logits = jnp.dot(pooled, wl_ref[...],
                     preferred_element_type=jnp.float32) + bl_ref[...]
    m = jnp.max(logits, axis=1, keepdims=True)
    ex = jnp.exp(logits - m)
    out_ref[...] = (logits - m) - jnp.log(jnp.sum(ex, axis=1, keepdims=True))


_final_call = pl.pallas_call(
    _final_body,
    out_shape=jax.ShapeDtypeStruct((1, C), jnp.float32),
)


def kernel(features, edge_index, W1, b1, W2, b2, W3, b3, Wl, bl):
    ei = edge_index.astype(jnp.int32)
    npad_e = EPAD - E
    trash = N + (jnp.arange(npad_e, dtype=jnp.int32) % (NPAD - N))
    src = jnp.concatenate([ei[0], trash]).reshape(NW, NBLK, BW)
    dst = jnp.concatenate([ei[1], trash]).reshape(NW, NBLK, BW)
    eidx = jnp.stack([src, dst], axis=2)
    x_p = jnp.pad(features, ((0, NPAD - N), (0, 0)))
    zeros = jnp.zeros((NPAD, F), jnp.float32)
    onehot0 = jnp.zeros((BW, F), jnp.float32).at[:, 0].set(1.0)
    onehot1 = jnp.zeros((BW, F), jnp.float32).at[:, 1].set(1.0)
    flag0 = jnp.zeros((16,), jnp.int32)
    flag1 = jnp.ones((16,), jnp.int32)

    deg = _agg_kernel(zeros, eidx, onehot0, onehot1, zeros, flag1)
    nsb, ndb = _prep_call(deg)

    s1 = _l1_call(x_p, W1, nsb)
    p1 = _agg_kernel(s1, eidx, onehot0, onehot1, zeros, flag0)
    s2 = _layer_call(p1, ndb, b1.reshape(1, F), W2, nsb)
    p2 = _agg_kernel(s2, eidx, onehot0, onehot1, zeros, flag0)
    s3 = _layer_call(p2, ndb, b2.reshape(1, F), W3, nsb)
    p3 = _agg_kernel(s3, eidx, onehot0, onehot1, zeros, flag0)
    return _final_call(p3, ndb, b3.reshape(1, F), Wl, bl.reshape(1, C))


# speedup vs baseline: 39.4355x; 1.1123x over previous
import functools

import jax
import jax.numpy as jnp
from jax import lax
from jax.experimental import pallas as pl
from jax.experimental.pallas import tpu as pltpu
from jax.experimental.pallas import tpu_sc as plsc

N = 10000
NPAD = 10112
E = 320000
NBLK = 80
BW = 128
NW = 32
EPAD = NW * NBLK * BW
F = 128
C = 32
ROWS_PER_SUB = NPAD // 16

_MESH = plsc.VectorSubcoreMesh(core_axis_name="c", subcore_axis_name="s")


@functools.partial(
    pl.kernel,
    out_type=jax.ShapeDtypeStruct((2, NPAD, F), jnp.float32),
    mesh=_MESH,
    scratch_types=[
        pltpu.VMEM((4, 2, BW), jnp.int32),
        pltpu.VMEM((BW, F), jnp.float32),
        pltpu.VMEM((BW, F), jnp.float32),
        pltpu.VMEM((16,), jnp.int32),
        pltpu.VMEM_SHARED((NPAD, F), jnp.float32),
        pltpu.SemaphoreType.DMA,
        pltpu.SemaphoreType.DMA,
        pltpu.SemaphoreType.DMA,
        pltpu.SemaphoreType.DMA,
        pltpu.SemaphoreType.DMA,
        pltpu.SemaphoreType.DMA,
        pltpu.SemaphoreType.DMA,
        pltpu.SemaphoreType.DMA,
    ],
)
def _agg_kernel(s_hbm, eidx, va_hbm, vb_hbm, zeros, flag, out,
                ibuf, rows0, rows1, flagv, acc,
                semi0, semi1, semi2, semi3, sem0, sem1, semw0, semw1):
    c = lax.axis_index("c")
    s = lax.axis_index("s")
    wid = s * 2 + c
    base = s * ROWS_PER_SUB
    semi = (semi0, semi1, semi2, semi3)
    gsem = (sem0, sem1)
    wsem = (semw0, semw1)
    rows = (rows0, rows1)
    pltpu.sync_copy(zeros.at[pl.ds(base, ROWS_PER_SUB)],
                    acc.at[pl.ds(base, ROWS_PER_SUB)])
    pltpu.sync_copy(flag, flagv)
    mode = flagv[...][0]
    plsc.subcore_barrier()

    def idx_start(j, slot):
        pltpu.async_copy(eidx.at[wid, j], ibuf.at[slot], semi[slot])

    def idx_wait(j, slot):
        pltpu.make_async_copy(eidx.at[wid, j], ibuf.at[slot], semi[slot]).wait()

    def g_start(slot, r):
        pltpu.async_copy(s_hbm.at[ibuf.at[slot, 0]], rows[r], gsem[r])

    def g_wait(slot, r):
        pltpu.make_async_copy(s_hbm.at[ibuf.at[slot, 0]], rows[r],
                              gsem[r]).wait()

    def w_start(slot, r):
        pltpu.async_copy(rows[r], acc.at[ibuf.at[slot, 1]], wsem[r], add=True)

    def w_wait(slot, r):
        pltpu.make_async_copy(rows[r], acc.at[ibuf.at[slot, 1]],
                              wsem[r]).wait()

    idx_start(0, 0)
    idx_start(1, 1)

    NQ = NBLK // 4

    @pl.when(mode == 0)
    def _layer():
        idx_wait(0, 0)
        g_start(0, 0)

        def body(q, carry):
            for b in range(4):
                r = b % 2
                pr = (b + 1) % 2
                slot_m1 = (b - 1) % 4
                slot_p2 = (b + 2) % 4

                if b == 0:
                    @pl.when(q > 0)
                    def _(q=q):
                        w_wait(0, 0)
                    idx_start(4 * q + 2, slot_p2)

                    @pl.when(q > 0)
                    def _(q=q):
                        idx_wait(4 * q, 0)
                        g_start(0, 0)
                elif b == 1:
                    @pl.when(q > 0)
                    def _(q=q):
                        w_wait(1, 1)
                    idx_start(4 * q + 3, slot_p2)
                    idx_wait(4 * q + 1, 1)
                    g_start(1, 1)
                else:
                    w_wait(b, r)

                    @pl.when(q < NQ - 1)
                    def _(q=q, b=b, slot_p2=slot_p2):
                        idx_start(4 * q + b + 2, slot_p2)

                    idx_wait(4 * q + b, b)
                    g_start(b, r)

                if b == 0:
                    @pl.when(q > 0)
                    def _():
                        g_wait(slot_m1, pr)
                        w_start(slot_m1, pr)
                else:
                    g_wait(slot_m1, pr)
                    w_start(slot_m1, pr)
            return carry

        lax.fori_loop(0, NQ, body, 0)
        g_wait(3, 1)
        w_wait(2, 0)
        w_start(3, 1)
        w_wait(3, 1)

    @pl.when(mode == 1)
    def _deg():
        pltpu.sync_copy(va_hbm, rows0)
        pltpu.sync_copy(vb_hbm, rows1)

        def scat(slot, r):
            pltpu.async_copy(rows0, acc.at[ibuf.at[slot, 0]],
                             wsem[r], add=True)
            pltpu.async_copy(rows1, acc.at[ibuf.at[slot, 1]],
                             gsem[r], add=True)

        def drain(slot, r):
            pltpu.make_async_copy(rows0, acc.at[ibuf.at[slot, 0]],
                                  wsem[r]).wait()
            pltpu.make_async_copy(rows1, acc.at[ibuf.at[slot, 1]],
                                  gsem[r]).wait()

        def body(q, carry):
            for b in range(4):
                r = b % 2
                slot_p2 = (b + 2) % 4

                if b < 2:
                    @pl.when(q > 0)
                    def _(q=q, b=b, r=r):
                        drain(b, r)
                    idx_start(4 * q + b + 2, slot_p2)
                    idx_wait(4 * q + b, b)
                    scat(b, r)
                else:
                    drain(b, r)

                    @pl.when(q < NQ - 1)
                    def _(q=q, b=b, slot_p2=slot_p2):
                        idx_start(4 * q + b + 2, slot_p2)

                    idx_wait(4 * q + b, b)
                    scat(b, r)
            return carry

        lax.fori_loop(0, NQ, body, 0)
        drain(2, 0)
        drain(3, 1)

    plsc.subcore_barrier()
    pltpu.sync_copy(acc.at[pl.ds(base, ROWS_PER_SUB)],
                    out.at[c, pl.ds(base, ROWS_PER_SUB)])


def _ns_blk(deg_ref):
    d = deg_ref[0] + deg_ref[1]
    return lax.rsqrt(jnp.maximum(d[:, 0:1], 1.0))


def _nd_blk(deg_ref):
    d = deg_ref[0] + deg_ref[1]
    return lax.rsqrt(jnp.maximum(d[:, 1:2], 1.0))


def _l1_body(x_ref, deg_ref, w_ref, out_ref):
    out_ref[...] = jnp.dot(x_ref[...], w_ref[...],
                           preferred_element_type=jnp.float32) * _ns_blk(deg_ref)


_l1_call = pl.pallas_call(
    _l1_body,
    grid=(NPAD // 128,),
    in_specs=[
        pl.BlockSpec((128, F), lambda i: (i, 0)),
        pl.BlockSpec((2, 128, F), lambda i: (0, i, 0)),
        pl.BlockSpec((F, F), lambda i: (0, 0)),
    ],
    out_specs=pl.BlockSpec((128, F), lambda i: (i, 0)),
    out_shape=jax.ShapeDtypeStruct((NPAD, F), jnp.float32),
)


def _layer_body(p_ref, deg_ref, b_ref, w_ref, out_ref):
    h = jnp.maximum((p_ref[0] + p_ref[1]) * _nd_blk(deg_ref) + b_ref[...], 0.0)
    out_ref[...] = jnp.dot(h, w_ref[...],
                           preferred_element_type=jnp.float32) * _ns_blk(deg_ref)


_layer_call = pl.pallas_call(
    _layer_body,
    grid=(NPAD // 128,),
    in_specs=[
        pl.BlockSpec((2, 128, F), lambda i: (0, i, 0)),
        pl.BlockSpec((2, 128, F), lambda i: (0, i, 0)),
        pl.BlockSpec((1, F), lambda i: (0, 0)),
        pl.BlockSpec((F, F), lambda i: (0, 0)),
    ],
    out_specs=pl.BlockSpec((128, F), lambda i: (i, 0)),
    out_shape=jax.ShapeDtypeStruct((NPAD, F), jnp.float32),
)


def _final_body(p_ref, deg_ref, b_ref, wl_ref, bl_ref, out_ref):
    d = deg_ref[0] + deg_ref[1]
    nd = lax.rsqrt(jnp.maximum(d[:, 1:2], 1.0))
    h = jnp.maximum((p_ref[0] + p_ref[1]) * nd + b_ref[...], 0.0)
    row = lax.broadcasted_iota(jnp.int32, (NPAD, F), 0)
    h = jnp.where(row < N, h, 0.0)
    pooled = jnp.sum(h, axis=0, keepdims=True) * (1.0 / N)
    logits = jnp.dot(pooled, wl_ref[...],
                     preferred_element_type=jnp.float32) + bl_ref[...]
    m = jnp.max(logits, axis=1, keepdims=True)
    ex = jnp.exp(logits - m)
    out_ref[...] = (logits - m) - jnp.log(jnp.sum(ex, axis=1, keepdims=True))


_final_call = pl.pallas_call(
    _final_body,
    out_shape=jax.ShapeDtypeStruct((1, C), jnp.float32),
)


def kernel(features, edge_index, W1, b1, W2, b2, W3, b3, Wl, bl):
    ei = edge_index.astype(jnp.int32)
    npad_e = EPAD - E
    trash = N + (jnp.arange(npad_e, dtype=jnp.int32) % (NPAD - N))
    src = jnp.concatenate([ei[0], trash]).reshape(NW, NBLK, BW)
    dst = jnp.concatenate([ei[1], trash]).reshape(NW, NBLK, BW)
    eidx = jnp.stack([src, dst], axis=2)
    x_p = jnp.pad(features, ((0, NPAD - N), (0, 0)))
    zeros = jnp.zeros((NPAD, F), jnp.float32)
    onehot0 = jnp.zeros((BW, F), jnp.float32).at[:, 0].set(1.0)
    onehot1 = jnp.zeros((BW, F), jnp.float32).at[:, 1].set(1.0)
    flag0 = jnp.zeros((16,), jnp.int32)
    flag1 = jnp.ones((16,), jnp.int32)

    deg = _agg_kernel(zeros, eidx, onehot0, onehot1, zeros, flag1)

    s1 = _l1_call(x_p, deg, W1)
    p1 = _agg_kernel(s1, eidx, onehot0, onehot1, zeros, flag0)
    s2 = _layer_call(p1, deg, b1.reshape(1, F), W2)
    p2 = _agg_kernel(s2, eidx, onehot0, onehot1, zeros, flag0)
    s3 = _layer_call(p2, deg, b2.reshape(1, F), W3)
    p3 = _agg_kernel(s3, eidx, onehot0, onehot1, zeros, flag0)
    return _final_call(p3, deg, b3.reshape(1, F), Wl, bl.reshape(1, C))


# speedup vs baseline: 47.7282x; 1.2103x over previous
import functools

import jax
import jax.numpy as jnp
from jax import lax
from jax.experimental import pallas as pl
from jax.experimental.pallas import tpu as pltpu
from jax.experimental.pallas import tpu_sc as plsc

N = 10000
NPAD = 10112
E = 320000
NBLK = 80
BW = 128
NW = 32
EPAD = NW * NBLK * BW
F = 128
C = 32
ROWS_PER_SUB = NPAD // 16

_MESH = plsc.VectorSubcoreMesh(core_axis_name="c", subcore_axis_name="s")


@functools.partial(
    pl.kernel,
    out_type=jax.ShapeDtypeStruct((2, NPAD, F), jnp.float32),
    mesh=_MESH,
    scratch_types=[
        pltpu.VMEM((4, 2, BW), jnp.int32),
        pltpu.VMEM((BW, F), jnp.float32),
        pltpu.VMEM((BW, F), jnp.float32),
        pltpu.VMEM((16,), jnp.int32),
        pltpu.VMEM_SHARED((NPAD, F), jnp.float32),
        pltpu.SemaphoreType.DMA,
        pltpu.SemaphoreType.DMA,
        pltpu.SemaphoreType.DMA,
        pltpu.SemaphoreType.DMA,
        pltpu.SemaphoreType.DMA,
        pltpu.SemaphoreType.DMA,
        pltpu.SemaphoreType.DMA,
        pltpu.SemaphoreType.DMA,
    ],
)
def _agg_kernel(s_hbm, eidx, va_hbm, vb_hbm, zeros, flag, out,
                ibuf, rows0, rows1, flagv, acc,
                semi0, semi1, semi2, semi3, sem0, sem1, semw0, semw1):
    c = lax.axis_index("c")
    s = lax.axis_index("s")
    wid = s * 2 + c
    base = s * ROWS_PER_SUB
    semi = (semi0, semi1, semi2, semi3)
    gsem = (sem0, sem1)
    wsem = (semw0, semw1)
    rows = (rows0, rows1)
    pltpu.sync_copy(zeros.at[pl.ds(base, ROWS_PER_SUB)],
                    acc.at[pl.ds(base, ROWS_PER_SUB)])
    pltpu.sync_copy(flag, flagv)
    mode = flagv[...][0]
    plsc.subcore_barrier()

    def idx_start(j, slot):
        pltpu.async_copy(eidx.at[wid, j], ibuf.at[slot], semi[slot])

    def idx_wait(j, slot):
        pltpu.make_async_copy(eidx.at[wid, j], ibuf.at[slot], semi[slot]).wait()

    def g_start(slot, r):
        pltpu.async_copy(s_hbm.at[ibuf.at[slot, 0]], rows[r], gsem[r])

    def g_wait(slot, r):
        pltpu.make_async_copy(s_hbm.at[ibuf.at[slot, 0]], rows[r],
                              gsem[r]).wait()

    def w_start(slot, r):
        pltpu.async_copy(rows[r], acc.at[ibuf.at[slot, 1]], wsem[r], add=True)

    def w_wait(slot, r):
        pltpu.make_async_copy(rows[r], acc.at[ibuf.at[slot, 1]],
                              wsem[r]).wait()

    idx_start(0, 0)
    idx_start(1, 1)

    NQ = NBLK // 4

    @pl.when(mode == 0)
    def _layer():
        idx_wait(0, 0)
        g_start(0, 0)

        def body(q, carry):
            for b in range(4):
                r = b % 2
                pr = (b + 1) % 2
                slot_m1 = (b - 1) % 4
                slot_p2 = (b + 2) % 4

                if b == 0:
                    @pl.when(q > 0)
                    def _(q=q):
                        w_wait(0, 0)
                    idx_start(4 * q + 2, slot_p2)

                    @pl.when(q > 0)
                    def _(q=q):
                        idx_wait(4 * q, 0)
                        g_start(0, 0)
                elif b == 1:
                    @pl.when(q > 0)
                    def _(q=q):
                        w_wait(1, 1)
                    idx_start(4 * q + 3, slot_p2)
                    idx_wait(4 * q + 1, 1)
                    g_start(1, 1)
                else:
                    w_wait(b, r)

                    @pl.when(q < NQ - 1)
                    def _(q=q, b=b, slot_p2=slot_p2):
                        idx_start(4 * q + b + 2, slot_p2)

                    idx_wait(4 * q + b, b)
                    g_start(b, r)

                if b == 0:
                    @pl.when(q > 0)
                    def _():
                        g_wait(slot_m1, pr)
                        w_start(slot_m1, pr)
                else:
                    g_wait(slot_m1, pr)
                    w_start(slot_m1, pr)
            return carry

        lax.fori_loop(0, NQ, body, 0)
        g_wait(3, 1)
        w_wait(2, 0)
        w_start(3, 1)
        w_wait(3, 1)

    @pl.when(mode == 1)
    def _deg():
        pltpu.sync_copy(va_hbm, rows0)
        pltpu.sync_copy(vb_hbm, rows1)

        def scat(slot, r):
            pltpu.async_copy(rows0, acc.at[ibuf.at[slot, 0]],
                             wsem[r], add=True)
            pltpu.async_copy(rows1, acc.at[ibuf.at[slot, 1]],
                             gsem[r], add=True)

        def drain(slot, r):
            pltpu.make_async_copy(rows0, acc.at[ibuf.at[slot, 0]],
                                  wsem[r]).wait()
            pltpu.make_async_copy(rows1, acc.at[ibuf.at[slot, 1]],
                                  gsem[r]).wait()

        def body(q, carry):
            for b in range(4):
                r = b % 2
                slot_p2 = (b + 2) % 4

                if b < 2:
                    @pl.when(q > 0)
                    def _(q=q, b=b, r=r):
                        drain(b, r)
                    idx_start(4 * q + b + 2, slot_p2)
                    idx_wait(4 * q + b, b)
                    scat(b, r)
                else:
                    drain(b, r)

                    @pl.when(q < NQ - 1)
                    def _(q=q, b=b, slot_p2=slot_p2):
                        idx_start(4 * q + b + 2, slot_p2)

                    idx_wait(4 * q + b, b)
                    scat(b, r)
            return carry

        lax.fori_loop(0, NQ, body, 0)
        drain(2, 0)
        drain(3, 1)

    plsc.subcore_barrier()
    pltpu.sync_copy(acc.at[pl.ds(base, ROWS_PER_SUB)],
                    out.at[c, pl.ds(base, ROWS_PER_SUB)])


def _ns_blk(deg_ref):
    d = deg_ref[0] + deg_ref[1]
    return lax.rsqrt(jnp.maximum(d[:, 0:1], 1.0))


def _nd_blk(deg_ref):
    d = deg_ref[0] + deg_ref[1]
    return lax.rsqrt(jnp.maximum(d[:, 1:2], 1.0))


def _l1_body(x_ref, deg_ref, w_ref, out_ref):
    out_ref[...] = jnp.dot(x_ref[...], w_ref[...],
                           preferred_element_type=jnp.float32) * _ns_blk(deg_ref)


_l1_call = pl.pallas_call(
    _l1_body,
    out_shape=jax.ShapeDtypeStruct((NPAD, F), jnp.float32),
)


def _layer_body(p_ref, deg_ref, b_ref, w_ref, out_ref):
    h = jnp.maximum((p_ref[0] + p_ref[1]) * _nd_blk(deg_ref) + b_ref[...], 0.0)
    out_ref[...] = jnp.dot(h, w_ref[...],
                           preferred_element_type=jnp.float32) * _ns_blk(deg_ref)


_layer_call = pl.pallas_call(
    _layer_body,
    out_shape=jax.ShapeDtypeStruct((NPAD, F), jnp.float32),
)


def _final_body(p_ref, deg_ref, b_ref, wl_ref, bl_ref, out_ref):
    d = deg_ref[0] + deg_ref[1]
    nd = lax.rsqrt(jnp.maximum(d[:, 1:2], 1.0))
    h = jnp.maximum((p_ref[0] + p_ref[1]) * nd + b_ref[...], 0.0)
    row = lax.broadcasted_iota(jnp.int32, (NPAD, F), 0)
    h = jnp.where(row < N, h, 0.0)
    pooled = jnp.sum(h, axis=0, keepdims=True) * (1.0 / N)
    logits = jnp.dot(pooled, wl_ref[...],
                     preferred_element_type=jnp.float32) + bl_ref[...]
    m = jnp.max(logits, axis=1, keepdims=True)
    ex = jnp.exp(logits - m)
    out_ref[...] = (logits - m) - jnp.log(jnp.sum(ex, axis=1, keepdims=True))


_final_call = pl.pallas_call(
    _final_body,
    out_shape=jax.ShapeDtypeStruct((1, C), jnp.float32),
)


def kernel(features, edge_index, W1, b1, W2, b2, W3, b3, Wl, bl):
    ei = edge_index.astype(jnp.int32)
    npad_e = EPAD - E
    trash = N + (jnp.arange(npad_e, dtype=jnp.int32) % (NPAD - N))
    src = jnp.concatenate([ei[0], trash]).reshape(NW, NBLK, BW)
    dst = jnp.concatenate([ei[1], trash]).reshape(NW, NBLK, BW)
    eidx = jnp.stack([src, dst], axis=2)
    x_p = jnp.pad(features, ((0, NPAD - N), (0, 0)))
    zeros = jnp.zeros((NPAD, F), jnp.float32)
    onehot0 = jnp.zeros((BW, F), jnp.float32).at[:, 0].set(1.0)
    onehot1 = jnp.zeros((BW, F), jnp.float32).at[:, 1].set(1.0)
    flag0 = jnp.zeros((16,), jnp.int32)
    flag1 = jnp.ones((16,), jnp.int32)

    deg = _agg_kernel(zeros, eidx, onehot0, onehot1, zeros, flag1)

    s1 = _l1_call(x_p, deg, W1)
    p1 = _agg_kernel(s1, eidx, onehot0, onehot1, zeros, flag0)
    s2 = _layer_call(p1, deg, b1.reshape(1, F), W2)
    p2 = _agg_kernel(s2, eidx, onehot0, onehot1, zeros, flag0)
    s3 = _layer_call(p2, deg, b2.reshape(1, F), W3)
    p3 = _agg_kernel(s3, eidx, onehot0, onehot1, zeros, flag0)
    return _final_call(p3, deg, b3.reshape(1, F), Wl, bl.reshape(1, C))
